# baseline (speedup 1.0000x reference)
import functools
import math

import jax
import jax.numpy as jnp
from jax import lax
from jax.experimental import pallas as pl
from jax.experimental.pallas import tpu as pltpu
from jax.experimental.pallas import tpu_sc as plsc

NUM_CLASS = 1000
LOW_DIM = 128
MOCO_QUEUE = 32768
BATCH = 1024
INV_T = 10.0
PROTO_M = 0.999
LN_M = math.log(PROTO_M)

BLK = 2048
NBLK = MOCO_QUEUE // BLK
NSTEP = NBLK + 1

NWORK = 32
ROWS_PW = LOW_DIM // NWORK
TAIL = MOCO_QUEUE - BATCH


def _main_body(q_ref, k_ref, qb_ref, logits_ref, carry_ref):
    j = pl.program_id(0)
    qs = q_ref[...] * INV_T
    qb = qb_ref[...]

    Qs = jnp.concatenate([carry_ref[...], qb[:, : BLK - 1]], axis=1)
    out = jnp.dot(qs, Qs, preferred_element_type=jnp.float32)
    carry_ref[...] = qb[:, BLK - 1:]

    @pl.when(j == 0)
    def _():
        lpos = jnp.sum(qs * k_ref[...], axis=1, keepdims=True)
        col = lax.broadcasted_iota(jnp.int32, (BATCH, BLK), 1)
        logits_ref[...] = jnp.where(col == 0, lpos, out)

    @pl.when(j > 0)
    def _():
        logits_ref[...] = out


def _lproto_body(q_ref, protos_ref, out_ref):
    out_ref[...] = lax.dot_general(
        q_ref[...] * INV_T, protos_ref[...], (((1,), (1,)), ((), ())),
        preferred_element_type=jnp.float32)


def _proto_body(protos_ref, q_ref, trow_ref, tcol_ref, out_ref):
    t = trow_ref[...]
    tc = tcol_ref[...]
    eq = (tc == t)
    ii = lax.broadcasted_iota(jnp.int32, (BATCH, BATCH), 0)
    jj = lax.broadcasted_iota(jnp.int32, (BATCH, BATCH), 1)
    pred = jnp.where(eq & (ii <= jj), 1.0, 0.0)
    both = jnp.where(eq, 1.0, 0.0)
    rank = jnp.sum(pred, axis=0, keepdims=True)
    cnt = jnp.sum(both, axis=0, keepdims=True)
    suffix = cnt - rank
    w = (1.0 - PROTO_M) * jnp.exp(suffix * LN_M)

    cls = lax.broadcasted_iota(jnp.int32, (NUM_CLASS, BATCH), 0)
    onehot = jnp.where(cls == t, 1.0, 0.0)
    hist = jnp.sum(onehot, axis=1, keepdims=True)
    decay = jnp.exp(hist * LN_M)

    upd = jnp.dot(onehot * w, q_ref[...],
                  preferred_element_type=jnp.float32)
    newp = decay * protos_ref[...] + upd
    norm = jnp.sqrt(jnp.sum(newp * newp, axis=1, keepdims=True))
    out_ref[...] = newp / jnp.maximum(norm, 1e-12)


_SC_MESH = plsc.VectorSubcoreMesh(core_axis_name="c", subcore_axis_name="s")


NCHUNK = 4
CHUNK = TAIL // NCHUNK


@functools.partial(
    pl.kernel,
    mesh=_SC_MESH,
    out_type=jax.ShapeDtypeStruct((LOW_DIM, MOCO_QUEUE), jnp.float32),
    scratch_types=[
        pltpu.VMEM((ROWS_PW, CHUNK), jnp.float32),
        pltpu.VMEM((ROWS_PW, CHUNK), jnp.float32),
        pltpu.SemaphoreType.DMA,
        pltpu.SemaphoreType.DMA,
    ],
)
def _sc_tail(queue_hbm, out_hbm, tba, tbb, sem_a, sem_b):
    wid = lax.axis_index("s") * 2 + lax.axis_index("c")
    base = wid * ROWS_PW
    rows = pl.ds(base, ROWS_PW)
    bufs = (tba, tbb)
    sems = (sem_a, sem_b)
    in_cp = [None, None]
    out_cp = [None, None]
    for ch in range(NCHUNK):
        b = ch % 2
        src = queue_hbm.at[rows, pl.ds(BATCH + ch * CHUNK, CHUNK)]
        if out_cp[b] is not None:
            out_cp[b].wait()
        in_cp[b] = pltpu.make_async_copy(src, bufs[b], sems[b])
        in_cp[b].start()
        in_cp[b].wait()
        dst = out_hbm.at[rows, pl.ds(BATCH + ch * CHUNK, CHUNK)]
        out_cp[b] = pltpu.make_async_copy(bufs[b], dst, sems[b])
        out_cp[b].start()
    for b in range(2):
        if out_cp[b] is not None:
            out_cp[b].wait()


def _head_body(k_ref, nq_ref, out_ref):
    del nq_ref
    out_ref[...] = k_ref[...].T


@functools.partial(jax.jit, static_argnames=())
def kernel(output, q, k, queue, prototypes, target):
    nq_tail = _sc_tail(queue)

    new_queue = pl.pallas_call(
        _head_body,
        grid=(1,),
        in_specs=[
            pl.BlockSpec((BATCH, LOW_DIM), lambda i: (0, 0)),
            pl.BlockSpec(memory_space=pl.ANY),
        ],
        out_specs=pl.BlockSpec((LOW_DIM, BATCH), lambda i: (0, 0)),
        out_shape=jax.ShapeDtypeStruct((LOW_DIM, MOCO_QUEUE), jnp.float32),
        input_output_aliases={1: 0},
    )(k, nq_tail)

    logits = pl.pallas_call(
        _main_body,
        grid=(NSTEP,),
        in_specs=[
            pl.BlockSpec((BATCH, LOW_DIM), lambda j: (0, 0)),
            pl.BlockSpec((BATCH, LOW_DIM), lambda j: (0, 0)),
            pl.BlockSpec((LOW_DIM, BLK), lambda j: (0, jnp.minimum(j, NBLK - 1))),
        ],
        out_specs=pl.BlockSpec((BATCH, BLK), lambda j: (0, j)),
        out_shape=jax.ShapeDtypeStruct((BATCH, MOCO_QUEUE + 1), jnp.float32),
        scratch_shapes=[pltpu.VMEM((LOW_DIM, 1), jnp.float32)],
        compiler_params=pltpu.CompilerParams(
            dimension_semantics=("arbitrary",)),
    )(q, k, queue)

    logits_proto = pl.pallas_call(
        _lproto_body,
        in_specs=[
            pl.BlockSpec((BATCH, LOW_DIM), lambda: (0, 0)),
            pl.BlockSpec((NUM_CLASS, LOW_DIM), lambda: (0, 0)),
        ],
        out_specs=pl.BlockSpec((BATCH, NUM_CLASS), lambda: (0, 0)),
        out_shape=jax.ShapeDtypeStruct((BATCH, NUM_CLASS), jnp.float32),
    )(q, prototypes)

    new_prototypes = pl.pallas_call(
        _proto_body,
        in_specs=[
            pl.BlockSpec((NUM_CLASS, LOW_DIM), lambda: (0, 0)),
            pl.BlockSpec((BATCH, LOW_DIM), lambda: (0, 0)),
            pl.BlockSpec((1, BATCH), lambda: (0, 0)),
            pl.BlockSpec((BATCH, 1), lambda: (0, 0)),
        ],
        out_specs=pl.BlockSpec((NUM_CLASS, LOW_DIM), lambda: (0, 0)),
        out_shape=jax.ShapeDtypeStruct((NUM_CLASS, LOW_DIM), jnp.float32),
    )(prototypes, q, target.reshape(1, BATCH), target.reshape(BATCH, 1))

    inst_labels = jnp.zeros((BATCH,), dtype=jnp.int32)
    return (output, target, logits, inst_labels, logits_proto,
            new_queue, new_prototypes)


# speedup vs baseline: 1.0078x; 1.0078x over previous
import functools
import math

import jax
import jax.numpy as jnp
from jax import lax
from jax.experimental import pallas as pl
from jax.experimental.pallas import tpu as pltpu
from jax.experimental.pallas import tpu_sc as plsc

NUM_CLASS = 1000
LOW_DIM = 128
MOCO_QUEUE = 32768
BATCH = 1024
INV_T = 10.0
PROTO_M = 0.999
LN_M = math.log(PROTO_M)

BLK = 4096
NBLK = MOCO_QUEUE // BLK
NSTEP = NBLK + 1

NWORK = 32
ROWS_PW = LOW_DIM // NWORK
TAIL = MOCO_QUEUE - BATCH


def _main_body(q_ref, k_ref, qb_ref, logits_ref, carry_ref):
    j = pl.program_id(0)
    qs = q_ref[...] * INV_T
    qb = qb_ref[...]

    Qs = jnp.concatenate([carry_ref[...], qb[:, : BLK - 1]], axis=1)
    out = jnp.dot(qs, Qs, preferred_element_type=jnp.float32)
    carry_ref[...] = qb[:, BLK - 1:]

    @pl.when(j == 0)
    def _():
        lpos = jnp.sum(qs * k_ref[...], axis=1, keepdims=True)
        col = lax.broadcasted_iota(jnp.int32, (BATCH, BLK), 1)
        logits_ref[...] = jnp.where(col == 0, lpos, out)

    @pl.when(j > 0)
    def _():
        logits_ref[...] = out


def _lproto_body(q_ref, protos_ref, out_ref):
    out_ref[...] = lax.dot_general(
        q_ref[...] * INV_T, protos_ref[...], (((1,), (1,)), ((), ())),
        preferred_element_type=jnp.float32)


def _proto_body(protos_ref, q_ref, trow_ref, tcol_ref, out_ref):
    t = trow_ref[...]
    tc = tcol_ref[...]
    eq = (tc == t)
    ii = lax.broadcasted_iota(jnp.int32, (BATCH, BATCH), 0)
    jj = lax.broadcasted_iota(jnp.int32, (BATCH, BATCH), 1)
    pred = jnp.where(eq & (ii <= jj), 1.0, 0.0)
    both = jnp.where(eq, 1.0, 0.0)
    rank = jnp.sum(pred, axis=0, keepdims=True)
    cnt = jnp.sum(both, axis=0, keepdims=True)
    suffix = cnt - rank
    w = (1.0 - PROTO_M) * jnp.exp(suffix * LN_M)

    cls = lax.broadcasted_iota(jnp.int32, (NUM_CLASS, BATCH), 0)
    onehot = jnp.where(cls == t, 1.0, 0.0)
    hist = jnp.sum(onehot, axis=1, keepdims=True)
    decay = jnp.exp(hist * LN_M)

    upd = jnp.dot(onehot * w, q_ref[...],
                  preferred_element_type=jnp.float32)
    newp = decay * protos_ref[...] + upd
    norm = jnp.sqrt(jnp.sum(newp * newp, axis=1, keepdims=True))
    out_ref[...] = newp / jnp.maximum(norm, 1e-12)


_SC_MESH = plsc.VectorSubcoreMesh(core_axis_name="c", subcore_axis_name="s")


NCHUNK = 4
CHUNK = TAIL // NCHUNK


@functools.partial(
    pl.kernel,
    mesh=_SC_MESH,
    out_type=jax.ShapeDtypeStruct((LOW_DIM, MOCO_QUEUE), jnp.float32),
    scratch_types=[
        pltpu.VMEM((ROWS_PW, CHUNK), jnp.float32),
        pltpu.VMEM((ROWS_PW, CHUNK), jnp.float32),
        pltpu.SemaphoreType.DMA,
        pltpu.SemaphoreType.DMA,
    ],
)
def _sc_tail(queue_hbm, out_hbm, tba, tbb, sem_a, sem_b):
    wid = lax.axis_index("s") * 2 + lax.axis_index("c")
    base = wid * ROWS_PW
    rows = pl.ds(base, ROWS_PW)
    bufs = (tba, tbb)
    sems = (sem_a, sem_b)
    in_cp = [None, None]
    out_cp = [None, None]
    for ch in range(NCHUNK):
        b = ch % 2
        src = queue_hbm.at[rows, pl.ds(BATCH + ch * CHUNK, CHUNK)]
        if out_cp[b] is not None:
            out_cp[b].wait()
        in_cp[b] = pltpu.make_async_copy(src, bufs[b], sems[b])
        in_cp[b].start()
        in_cp[b].wait()
        dst = out_hbm.at[rows, pl.ds(BATCH + ch * CHUNK, CHUNK)]
        out_cp[b] = pltpu.make_async_copy(bufs[b], dst, sems[b])
        out_cp[b].start()
    for b in range(2):
        if out_cp[b] is not None:
            out_cp[b].wait()


def _head_body(k_ref, nq_ref, out_ref):
    del nq_ref
    out_ref[...] = k_ref[...].T


@functools.partial(jax.jit, static_argnames=())
def kernel(output, q, k, queue, prototypes, target):
    nq_tail = _sc_tail(queue)

    new_queue = pl.pallas_call(
        _head_body,
        grid=(1,),
        in_specs=[
            pl.BlockSpec((BATCH, LOW_DIM), lambda i: (0, 0)),
            pl.BlockSpec(memory_space=pl.ANY),
        ],
        out_specs=pl.BlockSpec((LOW_DIM, BATCH), lambda i: (0, 0)),
        out_shape=jax.ShapeDtypeStruct((LOW_DIM, MOCO_QUEUE), jnp.float32),
        input_output_aliases={1: 0},
    )(k, nq_tail)

    logits = pl.pallas_call(
        _main_body,
        grid=(NSTEP,),
        in_specs=[
            pl.BlockSpec((BATCH, LOW_DIM), lambda j: (0, 0)),
            pl.BlockSpec((BATCH, LOW_DIM), lambda j: (0, 0)),
            pl.BlockSpec((LOW_DIM, BLK), lambda j: (0, jnp.minimum(j, NBLK - 1))),
        ],
        out_specs=pl.BlockSpec((BATCH, BLK), lambda j: (0, j)),
        out_shape=jax.ShapeDtypeStruct((BATCH, MOCO_QUEUE + 1), jnp.float32),
        scratch_shapes=[pltpu.VMEM((LOW_DIM, 1), jnp.float32)],
        compiler_params=pltpu.CompilerParams(
            dimension_semantics=("arbitrary",)),
    )(q, k, queue)

    logits_proto = pl.pallas_call(
        _lproto_body,
        in_specs=[
            pl.BlockSpec((BATCH, LOW_DIM), lambda: (0, 0)),
            pl.BlockSpec((NUM_CLASS, LOW_DIM), lambda: (0, 0)),
        ],
        out_specs=pl.BlockSpec((BATCH, NUM_CLASS), lambda: (0, 0)),
        out_shape=jax.ShapeDtypeStruct((BATCH, NUM_CLASS), jnp.float32),
    )(q, prototypes)

    new_prototypes = pl.pallas_call(
        _proto_body,
        in_specs=[
            pl.BlockSpec((NUM_CLASS, LOW_DIM), lambda: (0, 0)),
            pl.BlockSpec((BATCH, LOW_DIM), lambda: (0, 0)),
            pl.BlockSpec((1, BATCH), lambda: (0, 0)),
            pl.BlockSpec((BATCH, 1), lambda: (0, 0)),
        ],
        out_specs=pl.BlockSpec((NUM_CLASS, LOW_DIM), lambda: (0, 0)),
        out_shape=jax.ShapeDtypeStruct((NUM_CLASS, LOW_DIM), jnp.float32),
    )(prototypes, q, target.reshape(1, BATCH), target.reshape(BATCH, 1))

    inst_labels = jnp.zeros((BATCH,), dtype=jnp.int32)
    return (output, target, logits, inst_labels, logits_proto,
            new_queue, new_prototypes)


# speedup vs baseline: 1.0111x; 1.0033x over previous
import functools
import math

import jax
import jax.numpy as jnp
from jax import lax
from jax.experimental import pallas as pl
from jax.experimental.pallas import tpu as pltpu
from jax.experimental.pallas import tpu_sc as plsc

NUM_CLASS = 1000
LOW_DIM = 128
MOCO_QUEUE = 32768
BATCH = 1024
INV_T = 10.0
PROTO_M = 0.999
LN_M = math.log(PROTO_M)

BLK = 4096
NBLK = MOCO_QUEUE // BLK
NSTEP = NBLK + 1

NWORK = 32
ROWS_PW = LOW_DIM // NWORK
TAIL = MOCO_QUEUE - BATCH


def _main_body(q_ref, k_ref, qb_ref, logits_ref, carry_ref):
    j = pl.program_id(0)
    qs = q_ref[...] * INV_T
    qb = qb_ref[...]

    Qs = jnp.concatenate([carry_ref[...], qb[:, : BLK - 1]], axis=1)
    out = jnp.dot(qs, Qs, preferred_element_type=jnp.float32)
    carry_ref[...] = qb[:, BLK - 1:]

    @pl.when(j == 0)
    def _():
        lpos = jnp.sum(qs * k_ref[...], axis=1, keepdims=True)
        col = lax.broadcasted_iota(jnp.int32, (BATCH, BLK), 1)
        logits_ref[...] = jnp.where(col == 0, lpos, out)

    @pl.when(j > 0)
    def _():
        logits_ref[...] = out


def _lproto_body(q_ref, protos_ref, out_ref):
    out_ref[...] = lax.dot_general(
        q_ref[...] * INV_T, protos_ref[...], (((1,), (1,)), ((), ())),
        preferred_element_type=jnp.float32)


def _proto_body(protos_ref, q_ref, trow_ref, tcol_ref, out_ref):
    t = trow_ref[...]
    tc = tcol_ref[...]
    eq = (tc == t)
    ii = lax.broadcasted_iota(jnp.int32, (BATCH, BATCH), 0)
    jj = lax.broadcasted_iota(jnp.int32, (BATCH, BATCH), 1)
    pred = jnp.where(eq & (ii <= jj), 1.0, 0.0)
    both = jnp.where(eq, 1.0, 0.0)
    rank = jnp.sum(pred, axis=0, keepdims=True)
    cnt = jnp.sum(both, axis=0, keepdims=True)
    suffix = cnt - rank
    w = (1.0 - PROTO_M) * jnp.exp(suffix * LN_M)

    cls = lax.broadcasted_iota(jnp.int32, (NUM_CLASS, BATCH), 0)
    onehot = jnp.where(cls == t, 1.0, 0.0)
    hist = jnp.sum(onehot, axis=1, keepdims=True)
    decay = jnp.exp(hist * LN_M)

    upd = jnp.dot(onehot * w, q_ref[...],
                  preferred_element_type=jnp.float32)
    newp = decay * protos_ref[...] + upd
    norm = jnp.sqrt(jnp.sum(newp * newp, axis=1, keepdims=True))
    out_ref[...] = newp / jnp.maximum(norm, 1e-12)


_SC_MESH = plsc.VectorSubcoreMesh(core_axis_name="c", subcore_axis_name="s")


NCHUNK = 4
CHUNK = TAIL // NCHUNK


@functools.partial(
    pl.kernel,
    mesh=_SC_MESH,
    out_type=jax.ShapeDtypeStruct((LOW_DIM, MOCO_QUEUE), jnp.float32),
    scratch_types=[
        pltpu.VMEM((ROWS_PW, CHUNK), jnp.float32),
        pltpu.VMEM((ROWS_PW, CHUNK), jnp.float32),
        pltpu.SemaphoreType.DMA,
        pltpu.SemaphoreType.DMA,
    ],
)
def _sc_tail(queue_hbm, out_hbm, tba, tbb, sem_a, sem_b):
    wid = lax.axis_index("s") * 2 + lax.axis_index("c")
    base = wid * ROWS_PW
    rows = pl.ds(base, ROWS_PW)
    bufs = (tba, tbb)
    sems = (sem_a, sem_b)
    in_cp = [None, None]
    out_cp = [None, None]
    for ch in range(NCHUNK):
        b = ch % 2
        src = queue_hbm.at[rows, pl.ds(BATCH + ch * CHUNK, CHUNK)]
        if out_cp[b] is not None:
            out_cp[b].wait()
        in_cp[b] = pltpu.make_async_copy(src, bufs[b], sems[b])
        in_cp[b].start()
        in_cp[b].wait()
        dst = out_hbm.at[rows, pl.ds(BATCH + ch * CHUNK, CHUNK)]
        out_cp[b] = pltpu.make_async_copy(bufs[b], dst, sems[b])
        out_cp[b].start()
    for b in range(2):
        if out_cp[b] is not None:
            out_cp[b].wait()


def _head_body(k_ref, nq_ref, out_ref):
    del nq_ref
    out_ref[...] = k_ref[...].T


@functools.partial(jax.jit, static_argnames=())
def kernel(output, q, k, queue, prototypes, target):
    nq_tail = _sc_tail(queue)

    logits = pl.pallas_call(
        _main_body,
        grid=(NSTEP,),
        in_specs=[
            pl.BlockSpec((BATCH, LOW_DIM), lambda j: (0, 0)),
            pl.BlockSpec((BATCH, LOW_DIM), lambda j: (0, 0)),
            pl.BlockSpec((LOW_DIM, BLK), lambda j: (0, jnp.minimum(j, NBLK - 1))),
        ],
        out_specs=pl.BlockSpec((BATCH, BLK), lambda j: (0, j)),
        out_shape=jax.ShapeDtypeStruct((BATCH, MOCO_QUEUE + 1), jnp.float32),
        scratch_shapes=[pltpu.VMEM((LOW_DIM, 1), jnp.float32)],
        compiler_params=pltpu.CompilerParams(
            dimension_semantics=("arbitrary",)),
    )(q, k, queue)

    logits_proto = pl.pallas_call(
        _lproto_body,
        in_specs=[
            pl.BlockSpec((BATCH, LOW_DIM), lambda: (0, 0)),
            pl.BlockSpec((NUM_CLASS, LOW_DIM), lambda: (0, 0)),
        ],
        out_specs=pl.BlockSpec((BATCH, NUM_CLASS), lambda: (0, 0)),
        out_shape=jax.ShapeDtypeStruct((BATCH, NUM_CLASS), jnp.float32),
    )(q, prototypes)

    new_prototypes = pl.pallas_call(
        _proto_body,
        in_specs=[
            pl.BlockSpec((NUM_CLASS, LOW_DIM), lambda: (0, 0)),
            pl.BlockSpec((BATCH, LOW_DIM), lambda: (0, 0)),
            pl.BlockSpec((1, BATCH), lambda: (0, 0)),
            pl.BlockSpec((BATCH, 1), lambda: (0, 0)),
        ],
        out_specs=pl.BlockSpec((NUM_CLASS, LOW_DIM), lambda: (0, 0)),
        out_shape=jax.ShapeDtypeStruct((NUM_CLASS, LOW_DIM), jnp.float32),
    )(prototypes, q, target.reshape(1, BATCH), target.reshape(BATCH, 1))

    new_queue = pl.pallas_call(
        _head_body,
        grid=(1,),
        in_specs=[
            pl.BlockSpec((BATCH, LOW_DIM), lambda i: (0, 0)),
            pl.BlockSpec(memory_space=pl.ANY),
        ],
        out_specs=pl.BlockSpec((LOW_DIM, BATCH), lambda i: (0, 0)),
        out_shape=jax.ShapeDtypeStruct((LOW_DIM, MOCO_QUEUE), jnp.float32),
        input_output_aliases={1: 0},
    )(k, nq_tail)

    inst_labels = jnp.zeros((BATCH,), dtype=jnp.int32)
    return (output, target, logits, inst_labels, logits_proto,
            new_queue, new_prototypes)


# speedup vs baseline: 1.0477x; 1.0362x over previous
import functools
import math

import jax
import jax.numpy as jnp
from jax import lax
from jax.experimental import pallas as pl
from jax.experimental.pallas import tpu as pltpu
from jax.experimental.pallas import tpu_sc as plsc

NUM_CLASS = 1000
LOW_DIM = 128
MOCO_QUEUE = 32768
BATCH = 1024
INV_T = 10.0
PROTO_M = 0.999
LN_M = math.log(PROTO_M)

BLK = 2048
NBLK = MOCO_QUEUE // BLK
NSTEP = NBLK + 1

NSUB = 16
IPW = BATCH // NSUB
RPW = IPW
LAST_ROWS = NUM_CLASS - RPW * (NSUB - 1)


def _main_body(q_ref, k_ref, qb_ref, logits_ref, newq_ref, carry_ref):
    j = pl.program_id(0)
    qs = q_ref[...] * INV_T
    qb = qb_ref[...]

    Qs = jnp.concatenate([carry_ref[...], qb[:, : BLK - 1]], axis=1)
    out = jnp.dot(qs, Qs, preferred_element_type=jnp.float32)
    carry_ref[...] = qb[:, BLK - 1:]

    @pl.when(j == 0)
    def _():
        lpos = jnp.sum(qs * k_ref[...], axis=1, keepdims=True)
        col = lax.broadcasted_iota(jnp.int32, (BATCH, BLK), 1)
        logits_ref[...] = jnp.where(col == 0, lpos, out)
        newq_ref[:, :BATCH] = k_ref[...].T
        newq_ref[:, BATCH:] = qb[:, BATCH:]

    @pl.when(j > 0)
    def _():
        logits_ref[...] = out
        newq_ref[...] = qb


def _prep_body(q_ref, protos_ref, trow_ref, tcol_ref,
               lproto_ref, wq_ref, pdec_ref):
    qv = q_ref[...]
    lproto_ref[...] = lax.dot_general(
        qv * INV_T, protos_ref[...], (((1,), (1,)), ((), ())),
        preferred_element_type=jnp.float32)

    t = trow_ref[...]
    tc = tcol_ref[...]
    eq = (tc == t)
    ii = lax.broadcasted_iota(jnp.int32, (BATCH, BATCH), 0)
    jj = lax.broadcasted_iota(jnp.int32, (BATCH, BATCH), 1)
    rank = jnp.sum(jnp.where(eq & (jj <= ii), 1.0, 0.0),
                   axis=1, keepdims=True)
    cnt = jnp.sum(jnp.where(eq, 1.0, 0.0), axis=1, keepdims=True)
    w = (1.0 - PROTO_M) * jnp.exp((cnt - rank) * LN_M)
    wq_ref[...] = qv * w

    cls = lax.broadcasted_iota(jnp.int32, (NUM_CLASS, BATCH), 0)
    hist = jnp.sum(jnp.where(cls == t, 1.0, 0.0), axis=1, keepdims=True)
    pdec_ref[...] = jnp.exp(hist * LN_M) * protos_ref[...]


_SC_MESH = plsc.VectorSubcoreMesh(core_axis_name="c", subcore_axis_name="s")


@functools.partial(
    pl.kernel,
    mesh=_SC_MESH,
    out_type=jax.ShapeDtypeStruct((NUM_CLASS, LOW_DIM), jnp.float32),
    scratch_types=[
        pltpu.VMEM((IPW, LOW_DIM), jnp.float32),
        pltpu.VMEM((IPW,), jnp.int32),
        pltpu.VMEM_SHARED((NUM_CLASS, LOW_DIM), jnp.float32),
    ],
)
def _sc_ema(wq_hbm, tgt_hbm, pdec_hbm, out_hbm, wqbuf, idxbuf, acc):
    sid = lax.axis_index("s")

    @pl.when(lax.axis_index("c") == 0)
    def _():
        base = sid * RPW

        @pl.when(sid < NSUB - 1)
        def _():
            pltpu.sync_copy(pdec_hbm.at[pl.ds(base, RPW), :],
                            acc.at[pl.ds(base, RPW), :])

        @pl.when(sid == NSUB - 1)
        def _():
            pltpu.sync_copy(pdec_hbm.at[pl.ds(base, LAST_ROWS), :],
                            acc.at[pl.ds(base, LAST_ROWS), :])

        pltpu.sync_copy(tgt_hbm.at[pl.ds(sid * IPW, IPW)], idxbuf)
        pltpu.sync_copy(wq_hbm.at[pl.ds(sid * IPW, IPW), :], wqbuf)
        plsc.subcore_barrier()
        pltpu.sync_copy(wqbuf, acc.at[idxbuf], add=True)
        plsc.subcore_barrier()

        @pl.when(sid < NSUB - 1)
        def _():
            pltpu.sync_copy(acc.at[pl.ds(base, RPW), :],
                            out_hbm.at[pl.ds(base, RPW), :])

        @pl.when(sid == NSUB - 1)
        def _():
            pltpu.sync_copy(acc.at[pl.ds(base, LAST_ROWS), :],
                            out_hbm.at[pl.ds(base, LAST_ROWS), :])


def _finalize_body(s_ref, out_ref):
    s = s_ref[...]
    norm = jnp.sqrt(jnp.sum(s * s, axis=1, keepdims=True))
    out_ref[...] = s / jnp.maximum(norm, 1e-12)


@functools.partial(jax.jit, static_argnames=())
def kernel(output, q, k, queue, prototypes, target):
    logits, new_queue = pl.pallas_call(
        _main_body,
        grid=(NSTEP,),
        in_specs=[
            pl.BlockSpec((BATCH, LOW_DIM), lambda j: (0, 0)),
            pl.BlockSpec((BATCH, LOW_DIM), lambda j: (0, 0)),
            pl.BlockSpec((LOW_DIM, BLK), lambda j: (0, jnp.minimum(j, NBLK - 1))),
        ],
        out_specs=[
            pl.BlockSpec((BATCH, BLK), lambda j: (0, j)),
            pl.BlockSpec((LOW_DIM, BLK), lambda j: (0, jnp.minimum(j, NBLK - 1))),
        ],
        out_shape=[
            jax.ShapeDtypeStruct((BATCH, MOCO_QUEUE + 1), jnp.float32),
            jax.ShapeDtypeStruct((LOW_DIM, MOCO_QUEUE), jnp.float32),
        ],
        scratch_shapes=[pltpu.VMEM((LOW_DIM, 1), jnp.float32)],
        compiler_params=pltpu.CompilerParams(
            dimension_semantics=("arbitrary",)),
    )(q, k, queue)

    logits_proto, wq, pdec = pl.pallas_call(
        _prep_body,
        in_specs=[
            pl.BlockSpec((BATCH, LOW_DIM), lambda: (0, 0)),
            pl.BlockSpec((NUM_CLASS, LOW_DIM), lambda: (0, 0)),
            pl.BlockSpec((1, BATCH), lambda: (0, 0)),
            pl.BlockSpec((BATCH, 1), lambda: (0, 0)),
        ],
        out_specs=[
            pl.BlockSpec((BATCH, NUM_CLASS), lambda: (0, 0)),
            pl.BlockSpec((BATCH, LOW_DIM), lambda: (0, 0)),
            pl.BlockSpec((NUM_CLASS, LOW_DIM), lambda: (0, 0)),
        ],
        out_shape=[
            jax.ShapeDtypeStruct((BATCH, NUM_CLASS), jnp.float32),
            jax.ShapeDtypeStruct((BATCH, LOW_DIM), jnp.float32),
            jax.ShapeDtypeStruct((NUM_CLASS, LOW_DIM), jnp.float32),
        ],
    )(q, prototypes, target.reshape(1, BATCH), target.reshape(BATCH, 1))

    s_acc = _sc_ema(wq, target, pdec)

    new_prototypes = pl.pallas_call(
        _finalize_body,
        in_specs=[pl.BlockSpec((NUM_CLASS, LOW_DIM), lambda: (0, 0))],
        out_specs=pl.BlockSpec((NUM_CLASS, LOW_DIM), lambda: (0, 0)),
        out_shape=jax.ShapeDtypeStruct((NUM_CLASS, LOW_DIM), jnp.float32),
    )(s_acc)

    inst_labels = jnp.zeros((BATCH,), dtype=jnp.int32)
    return (output, target, logits, inst_labels, logits_proto,
            new_queue, new_prototypes)
